# speedup vs baseline: 1.6231x; 1.6231x over previous
import functools

import jax
import jax.numpy as jnp
from jax import lax
from jax.experimental import pallas as pl
from jax.experimental.pallas import tpu as pltpu
from jax.experimental.pallas import tpu_sc as plsc

_NC = 2
_NS = 16
_NW = _NC * _NS
_C = 8


@functools.lru_cache(maxsize=None)
def _build(n, v, d):
    assert n % (_NW * _C) == 0
    bpw = n // _NW
    nchunk = bpw // _C

    mesh = plsc.VectorSubcoreMesh(core_axis_name="c", subcore_axis_name="s")

    @functools.partial(
        pl.kernel,
        out_type=jax.ShapeDtypeStruct((n, d), jnp.float32),
        mesh=mesh,
        scratch_types=[
            pltpu.VMEM((bpw,), jnp.int32),
            pltpu.VMEM((_C, d), jnp.float32),
            pltpu.SemaphoreType.DMA,
        ],
    )
    def emb(idx_hbm, table_hbm, out_hbm, idx_v, rows_v, sem):
        wid = lax.axis_index("s") * _NC + lax.axis_index("c")
        base = wid * bpw
        pltpu.sync_copy(idx_hbm.at[pl.ds(base, bpw)], idx_v)

        def chunk(i, carry):
            off = i * _C
            pltpu.async_copy(
                table_hbm.at[idx_v.at[pl.ds(off, _C)]], rows_v, sem
            ).wait()
            pltpu.sync_copy(rows_v, out_hbm.at[pl.ds(base + off, _C)])
            return carry

        lax.fori_loop(0, nchunk, chunk, 0)

    return emb


def kernel(indices, embedding_weight):
    b, t = indices.shape
    v, d = embedding_weight.shape
    flat = indices.reshape(-1).astype(jnp.int32)
    out = _build(flat.shape[0], v, d)(flat, embedding_weight)
    return out.reshape(b, t, d)


# speedup vs baseline: 1.9881x; 1.2249x over previous
import functools

import jax
import jax.numpy as jnp
from jax import lax
from jax.experimental import pallas as pl
from jax.experimental.pallas import tpu as pltpu
from jax.experimental.pallas import tpu_sc as plsc

_NC = 2
_NS = 16
_NW = _NC * _NS
_C = 8


@functools.lru_cache(maxsize=None)
def _build(n, v, d):
    assert n % (_NW * 2 * _C) == 0
    bpw = n // _NW
    nchunk = bpw // _C
    half = nchunk // 2

    mesh = plsc.VectorSubcoreMesh(core_axis_name="c", subcore_axis_name="s")

    @functools.partial(
        pl.kernel,
        out_type=jax.ShapeDtypeStruct((n, d), jnp.float32),
        mesh=mesh,
        scratch_types=[
            pltpu.VMEM((bpw,), jnp.int32),
            pltpu.VMEM((_C, d), jnp.float32),
            pltpu.VMEM((_C, d), jnp.float32),
            pltpu.SemaphoreType.DMA,
            pltpu.SemaphoreType.DMA,
            pltpu.SemaphoreType.DMA,
            pltpu.SemaphoreType.DMA,
        ],
    )
    def emb(idx_hbm, table_hbm, out_hbm, idx_v, rows0, rows1, gs0, gs1,
            os0, os1):
        wid = lax.axis_index("s") * _NC + lax.axis_index("c")
        base = wid * bpw
        pltpu.sync_copy(idx_hbm.at[pl.ds(base, bpw)], idx_v)

        def gather(chunk, rows, sem):
            off = chunk * _C
            pltpu.async_copy(table_hbm.at[idx_v.at[pl.ds(off, _C)]], rows,
                             sem)

        def wait_gather(rows, sem):
            pltpu.make_async_copy(table_hbm.at[pl.ds(0, _C)], rows,
                                  sem).wait()

        def put(chunk, rows, sem):
            off = chunk * _C
            pltpu.async_copy(rows, out_hbm.at[pl.ds(base + off, _C)], sem)

        def wait_put(rows, sem):
            pltpu.make_async_copy(rows, out_hbm.at[pl.ds(base, _C)],
                                  sem).wait()

        gather(0, rows0, gs0)

        def body(g, carry):
            @pl.when(g > 0)
            def _():
                wait_put(rows1, os1)

            gather(2 * g + 1, rows1, gs1)
            wait_gather(rows0, gs0)
            put(2 * g, rows0, os0)

            @pl.when(g < half - 1)
            def _():
                wait_put(rows0, os0)
                gather(2 * g + 2, rows0, gs0)

            wait_gather(rows1, gs1)
            put(2 * g + 1, rows1, os1)
            return carry

        lax.fori_loop(0, half, body, 0)
        wait_put(rows0, os0)
        wait_put(rows1, os1)

    return emb


def kernel(indices, embedding_weight):
    b, t = indices.shape
    v, d = embedding_weight.shape
    flat = indices.reshape(-1).astype(jnp.int32)
    out = _build(flat.shape[0], v, d)(flat, embedding_weight)
    return out.reshape(b, t, d)
